# Initial kernel scaffold; baseline (speedup 1.0000x reference)
import jax
import jax.numpy as jnp
from jax.experimental import pallas as pl


def kernel(x, edge_index, W1, b1, W2, b2, lin_W, lin_b, fc_W, fc_b):
    raise NotImplementedError("write your pallas kernel here")


# speedup vs baseline: 11.0491x; 11.0491x over previous
import functools
import jax
import jax.numpy as jnp
from jax import lax
from jax.experimental import pallas as pl
from jax.experimental.pallas import tpu as pltpu
from jax.experimental.pallas import tpu_sc as plsc

N = 10000
E = 320000
D = 128
NOUT = 40
NC, NS = 2, 16
NW = NC * NS
K = 128
NP = 10240
ROWS_PER_SUB = NP // NS
EP = ((E + NW * K - 1) // (NW * K)) * (NW * K)
EPW = EP // NW
NITER = EPW // K
R = 2048
GRID = NP // R


def _deg_body(dst_hbm, out_hbm, idx_v, ones_v, zer_v, acc_sh, sem):
    c = lax.axis_index("c")
    s = lax.axis_index("s")
    wid = s * NC + c
    for j in range(K // 16):
        ones_v[pl.ds(j * 16, 16)] = jnp.ones((16,), jnp.float32)
    for j in range(ROWS_PER_SUB // 16):
        zer_v[pl.ds(j * 16, 16)] = jnp.zeros((16,), jnp.float32)
    pltpu.sync_copy(zer_v, acc_sh.at[pl.ds(s * ROWS_PER_SUB, ROWS_PER_SUB)])
    plsc.subcore_barrier()

    base = wid * EPW

    def body(i, carry):
        off = pl.multiple_of(base + i * K, 8)
        pltpu.sync_copy(dst_hbm.at[pl.ds(off, K)], idx_v)
        pltpu.sync_copy(ones_v, acc_sh.at[idx_v], add=True)
        return carry

    lax.fori_loop(0, NITER, body, 0)
    plsc.subcore_barrier()
    pltpu.sync_copy(acc_sh.at[pl.ds(s * ROWS_PER_SUB, ROWS_PER_SUB)],
                    out_hbm.at[c, pl.ds(s * ROWS_PER_SUB, ROWS_PER_SUB)])


_deg_kernel = functools.partial(
    pl.kernel,
    out_type=jax.ShapeDtypeStruct((NC, NP), jnp.float32),
    mesh=plsc.VectorSubcoreMesh(core_axis_name="c", subcore_axis_name="s"),
    scratch_types=[
        pltpu.VMEM((K,), jnp.int32),
        pltpu.VMEM((K,), jnp.float32),
        pltpu.VMEM((ROWS_PER_SUB,), jnp.float32),
        pltpu.VMEM_SHARED((NP,), jnp.float32),
        pltpu.SemaphoreType.DMA,
    ],
)(_deg_body)


def _prop_body(g_hbm, src_hbm, dst_hbm, out_hbm, src_v, dst_v, rows_v, acc_sh, sem):
    c = lax.axis_index("c")
    s = lax.axis_index("s")
    wid = s * NC + c

    def zbody(i, carry):
        for j in range(D // 16):
            rows_v[i, pl.ds(j * 16, 16)] = jnp.zeros((16,), jnp.float32)
        return carry

    lax.fori_loop(0, K, zbody, 0)
    for t in range(ROWS_PER_SUB // K):
        pltpu.sync_copy(rows_v, acc_sh.at[pl.ds(s * ROWS_PER_SUB + t * K, K)])
    plsc.subcore_barrier()

    base = wid * EPW

    def body(i, carry):
        off = pl.multiple_of(base + i * K, 8)
        pltpu.sync_copy(src_hbm.at[pl.ds(off, K)], src_v)
        pltpu.sync_copy(dst_hbm.at[pl.ds(off, K)], dst_v)
        pltpu.async_copy(g_hbm.at[src_v], rows_v, sem).wait()
        pltpu.sync_copy(rows_v, acc_sh.at[dst_v], add=True)
        return carry

    lax.fori_loop(0, NITER, body, 0)
    plsc.subcore_barrier()
    pltpu.sync_copy(acc_sh.at[pl.ds(s * ROWS_PER_SUB, ROWS_PER_SUB)],
                    out_hbm.at[c, pl.ds(s * ROWS_PER_SUB, ROWS_PER_SUB)])


_prop_kernel = functools.partial(
    pl.kernel,
    out_type=jax.ShapeDtypeStruct((NC, NP, D), jnp.float32),
    mesh=plsc.VectorSubcoreMesh(core_axis_name="c", subcore_axis_name="s"),
    scratch_types=[
        pltpu.VMEM((K,), jnp.int32),
        pltpu.VMEM((K,), jnp.int32),
        pltpu.VMEM((K, D), jnp.float32),
        pltpu.VMEM_SHARED((NP, D), jnp.float32),
        pltpu.SemaphoreType.DMA,
    ],
)(_prop_body)


def _dis_block(degp_ref):
    deg = degp_ref[0, :] + degp_ref[1, :] + 1.0
    return lax.rsqrt(deg)


def _valid_mask(i):
    rows = lax.broadcasted_iota(jnp.int32, (R, 1), 0) + i * R
    return rows < N


def _b1_body(x_ref, degp_ref, w_ref, g_ref):
    i = pl.program_id(0)
    dis = _dis_block(degp_ref)
    h = jnp.dot(x_ref[...], w_ref[...], preferred_element_type=jnp.float32)
    g_ref[...] = jnp.where(_valid_mask(i), dis[:, None] * h, 0.0)


def _b2_body(s_ref, g_ref, degp_ref, b_ref, w_ref, g2_ref, cs_ref):
    i = pl.program_id(0)
    dis = _dis_block(degp_ref)
    tot = s_ref[0] + s_ref[1] + g_ref[...]
    x1 = jnp.maximum(dis[:, None] * tot + b_ref[...], 0.0)
    x1 = jnp.where(_valid_mask(i), x1, 0.0)

    @pl.when(i == 0)
    def _():
        cs_ref[...] = jnp.zeros_like(cs_ref)

    cs_ref[...] += jnp.sum(x1, axis=0, keepdims=True)
    h2 = jnp.dot(x1, w_ref[...], preferred_element_type=jnp.float32)
    g2_ref[...] = dis[:, None] * h2


def _b3_body(s_ref, g_ref, degp_ref, b_ref, cs_ref):
    i = pl.program_id(0)
    dis = _dis_block(degp_ref)
    tot = s_ref[0] + s_ref[1] + g_ref[...]
    x2 = jnp.maximum(dis[:, None] * tot + b_ref[...], 0.0)
    x2 = jnp.where(_valid_mask(i), x2, 0.0)

    @pl.when(i == 0)
    def _():
        cs_ref[...] = jnp.zeros_like(cs_ref)

    cs_ref[...] += jnp.sum(x2, axis=0, keepdims=True)


def _b4_body(cs1_ref, cs2_ref, linw_ref, linb_ref, fcw_ref, fcb_ref, o_ref):
    m1 = cs1_ref[...] * (1.0 / N)
    m2 = cs2_ref[...] * (1.0 / N)
    pooled = (jnp.dot(m1, linw_ref[0:D, :], preferred_element_type=jnp.float32)
              + jnp.dot(m2, linw_ref[D:2 * D, :], preferred_element_type=jnp.float32)
              + linb_ref[...])
    logits = jnp.dot(pooled, fcw_ref[...], preferred_element_type=jnp.float32) + fcb_ref[...]
    e = jnp.exp(logits - jnp.max(logits, axis=1, keepdims=True))
    o_ref[...] = e / jnp.sum(e, axis=1, keepdims=True)


_spec_rows = pl.BlockSpec((R, D), lambda i: (i, 0))
_spec_degp = pl.BlockSpec((2, R), lambda i: (0, i))
_spec_w = pl.BlockSpec((D, D), lambda i: (0, 0))
_spec_b = pl.BlockSpec((1, D), lambda i: (0, 0))
_spec_s = pl.BlockSpec((2, R, D), lambda i: (0, i, 0))
_spec_cs = pl.BlockSpec((1, D), lambda i: (0, 0))

_b1_call = pl.pallas_call(
    _b1_body,
    grid=(GRID,),
    in_specs=[_spec_rows, _spec_degp, _spec_w],
    out_specs=_spec_rows,
    out_shape=jax.ShapeDtypeStruct((NP, D), jnp.float32),
)

_b2_call = pl.pallas_call(
    _b2_body,
    grid=(GRID,),
    in_specs=[_spec_s, _spec_rows, _spec_degp, _spec_b, _spec_w],
    out_specs=[_spec_rows, _spec_cs],
    out_shape=[jax.ShapeDtypeStruct((NP, D), jnp.float32),
               jax.ShapeDtypeStruct((1, D), jnp.float32)],
)

_b3_call = pl.pallas_call(
    _b3_body,
    grid=(GRID,),
    in_specs=[_spec_s, _spec_rows, _spec_degp, _spec_b],
    out_specs=_spec_cs,
    out_shape=jax.ShapeDtypeStruct((1, D), jnp.float32),
)

_b4_call = pl.pallas_call(
    _b4_body,
    in_specs=[pl.BlockSpec((1, D), lambda: (0, 0)),
              pl.BlockSpec((1, D), lambda: (0, 0)),
              pl.BlockSpec((2 * D, D), lambda: (0, 0)),
              pl.BlockSpec((1, D), lambda: (0, 0)),
              pl.BlockSpec((D, NOUT), lambda: (0, 0)),
              pl.BlockSpec((1, NOUT), lambda: (0, 0))],
    out_specs=pl.BlockSpec((1, NOUT), lambda: (0, 0)),
    out_shape=jax.ShapeDtypeStruct((1, NOUT), jnp.float32),
)


@jax.jit
def kernel(x, edge_index, W1, b1, W2, b2, lin_W, lin_b, fc_W, fc_b):
    src = edge_index[0].astype(jnp.int32)
    dst = edge_index[1].astype(jnp.int32)
    pad = EP - E
    srcp = jnp.concatenate([src, jnp.zeros((pad,), jnp.int32)])
    dstp = jnp.concatenate(
        [dst, N + (jnp.arange(pad, dtype=jnp.int32) % (NP - N))])

    degp = _deg_kernel(dstp)
    g1 = _b1_call(x, degp, W1)
    s1 = _prop_kernel(g1, srcp, dstp)
    g2, cs1 = _b2_call(s1, g1, degp, b1.reshape(1, D), W2)
    s2 = _prop_kernel(g2, srcp, dstp)
    cs2 = _b3_call(s2, g2, degp, b2.reshape(1, D))
    out = _b4_call(cs1, cs2, lin_W, lin_b.reshape(1, D),
                   fc_W, fc_b.reshape(1, NOUT))
    return out
